# baseline (speedup 1.0000x reference)
import jax
import jax.numpy as jnp
from jax import lax
from jax.experimental import pallas as pl
from jax.experimental.pallas import tpu as pltpu
from jax.experimental.pallas import tpu_sc as plsc

_NC, _NS = 2, 16
_NW = _NC * _NS

_EXP_CACHE = {}


def _std_exponential(shape):
    if shape not in _EXP_CACHE:
        e = jax.random.exponential(jax.random.key(1), shape, dtype=jnp.float32)
        _EXP_CACHE[shape] = e.reshape(shape[0], -1, shape[-1])
    return _EXP_CACHE[shape]


def _tc_reduce_block(r_ref, dt_ref, act_ref):
    gb, rows, lanes = r_ref.shape
    per = rows * lanes
    i = pl.program_id(0)
    lin = (lax.broadcasted_iota(jnp.int32, (gb, rows, lanes), 1) * lanes
           + lax.broadcasted_iota(jnp.int32, (gb, rows, lanes), 2))
    bi = lax.broadcasted_iota(jnp.int32, (gb, rows, lanes), 0)
    n = (i * gb + bi) * per + lin

    def rnds(a, b2, rs):
        for rr in rs:
            a = a + b2
            b2 = (b2 << rr) | (b2 >> (32 - rr))
            b2 = b2 ^ a
        return a, b2

    r1 = (13, 15, 26, 6)
    r2 = (17, 29, 16, 24)
    ks2 = 0x1BD11BDB
    a = jnp.zeros_like(n).astype(jnp.uint32)
    b2 = n.astype(jnp.uint32) + 1
    a, b2 = rnds(a, b2, r1); a = a + 1;   b2 = b2 + (ks2 + 1)
    a, b2 = rnds(a, b2, r2); a = a + ks2; b2 = b2 + 2
    a, b2 = rnds(a, b2, r1); a = a + 0;   b2 = b2 + (1 + 3)
    a, b2 = rnds(a, b2, r2); a = a + 1;   b2 = b2 + (ks2 + 4)
    a, b2 = rnds(a, b2, r1); a = a + ks2; b2 = b2 + (0 + 5)
    bits = a ^ b2

    ub = (bits >> 9) | jnp.uint32(0x3F800000)
    u = lax.bitcast_convert_type(ub, jnp.float32) - 1.0
    e = -jnp.log1p(-u)

    times = e / r_ref[...]
    minv = jnp.min(times, axis=(1, 2), keepdims=True)
    big = jnp.int32(2**30)
    act = jnp.min(jnp.where(times == minv, lin, big), axis=(1, 2))
    dt_ref[...] = minv.reshape(gb, 1)
    act_ref[...] = act.reshape(gb, 1)


def _tc_reduce(r):
    b, rows, ls = r.shape
    gb = 8
    dt, act = pl.pallas_call(
        _tc_reduce_block,
        grid=(b // gb,),
        in_specs=[
            pl.BlockSpec((gb, rows, ls), lambda i: (i, 0, 0)),
        ],
        out_specs=[
            pl.BlockSpec((gb, 1), lambda i: (i, 0)),
            pl.BlockSpec((gb, 1), lambda i: (i, 0)),
        ],
        out_shape=[
            jax.ShapeDtypeStruct((b, 1), jnp.float32),
            jax.ShapeDtypeStruct((b, 1), jnp.int32),
        ],
    )(r)
    return dt.reshape(b), act.reshape(b)


def _make_sc_copy(total):
    per_w = total // _NW
    chunk = 32768
    n = per_w // chunk
    mesh = plsc.VectorSubcoreMesh(core_axis_name="c", subcore_axis_name="s", num_cores=_NC, num_subcores=_NS)

    def body(s_hbm, y_hbm, buf, si0, si1, so0, so1):
        wid = lax.axis_index("s") * _NC + lax.axis_index("c")
        base = wid * per_w
        sems_in = (si0, si1)
        sems_out = (so0, so1)

        def in_copy(c):
            return pltpu.async_copy(
                s_hbm.at[pl.ds(base + c * chunk, chunk)], buf.at[c % 2],
                sems_in[c % 2])

        def out_copy(c):
            return pltpu.async_copy(
                buf.at[c % 2], y_hbm.at[pl.ds(base + c * chunk, chunk)],
                sems_out[c % 2])

        ins = [None] * n
        outs = [None] * n
        ins[0] = in_copy(0)
        for c in range(n):
            if c + 1 < n:
                if c >= 1:
                    outs[c - 1].wait()
                ins[c + 1] = in_copy(c + 1)
            ins[c].wait()
            outs[c] = out_copy(c)
        if n >= 2:
            outs[n - 2].wait()
        outs[n - 1].wait()

    return pl.kernel(
        body,
        out_type=jax.ShapeDtypeStruct((total,), jnp.float32),
        mesh=mesh,
        scratch_types=[
            pltpu.VMEM((2, chunk), jnp.float32),
            pltpu.SemaphoreType.DMA,
            pltpu.SemaphoreType.DMA,
            pltpu.SemaphoreType.DMA,
            pltpu.SemaphoreType.DMA,
        ],
    )


def _make_sc_flip(b, ls):
    pb = b // _NW
    mesh = plsc.VectorSubcoreMesh(core_axis_name="c", subcore_axis_name="s", num_cores=_NC, num_subcores=_NS)

    def body(act_hbm, s_hbm, y_hbm, act_v, idx_v, vals_v, sem):
        wid = lax.axis_index("s") * _NC + lax.axis_index("c")
        base_b = wid * pb
        pltpu.sync_copy(act_hbm.at[pl.ds(base_b, pb)], act_v)
        for j in range(pb // 16):
            a = act_v[pl.ds(j * 16, 16)]
            m = lax.rem(a, ls)
            t1 = lax.div(a, ls)
            l = lax.rem(t1, ls)
            d = lax.div(t1, ls)
            l2 = jnp.where(d == 0, jnp.where(l == 0, ls - 1, l - 1), l)
            m2 = jnp.where(d == 0, m, jnp.where(m == ls - 1, 0, m + 1))
            bvec = base_b + j * 16 + lax.iota(jnp.int32, 16)
            idx_v[pl.ds(j * 16, 16)] = (bvec * ls + l) * ls + m
            idx_v[pl.ds(pb + j * 16, 16)] = (bvec * ls + l2) * ls + m2
        pltpu.async_copy(s_hbm.at[idx_v], vals_v, sem).wait()
        for g in range(2 * pb // 16):
            x = vals_v[pl.ds(g * 16, 16)]
            vals_v[pl.ds(g * 16, 16)] = 1.0 - x
        pltpu.async_copy(vals_v, y_hbm.at[idx_v], sem).wait()

    return pl.kernel(
        body,
        out_type=(),
        mesh=mesh,
        scratch_types=[
            pltpu.VMEM((pb,), jnp.int32),
            pltpu.VMEM((2 * pb,), jnp.int32),
            pltpu.VMEM((2 * pb,), jnp.float32),
            pltpu.SemaphoreType.DMA,
        ],
    )


def kernel(state, rates, k):
    b, ls = state.shape[0], state.shape[-1]
    r = rates.reshape(b, -1, ls)

    y0 = _make_sc_copy(b * ls * ls)(state.reshape(b * ls * ls))
    dt, act = _tc_reduce(r)

    yref = jax.new_ref(y0)
    _make_sc_flip(b, ls)(act, state.reshape(b * ls * ls), yref)
    y = jax.freeze(yref).reshape(b, ls, ls)
    return (y, dt, act)
